# baseline (speedup 1.0000x reference)
import functools

import jax
import jax.numpy as jnp
from jax import lax
from jax.experimental import pallas as pl
from jax.experimental.pallas import tpu as pltpu
from jax.experimental.pallas import tpu_sc as plsc

_BATCH = 16384
_NC, _NS = 2, 16
_NW = _NC * _NS
_B_PER_W = _BATCH // _NW
_HALF = _B_PER_W // 2


def _make_gather():
    mesh = plsc.VectorSubcoreMesh(core_axis_name="c", subcore_axis_name="s")

    @functools.partial(
        pl.kernel,
        mesh=mesh,
        out_type=jax.ShapeDtypeStruct((_BATCH,), jnp.float32),
        scratch_types=[
            pltpu.VMEM((_B_PER_W,), jnp.int32),
            pltpu.VMEM((_B_PER_W,), jnp.float32),
            pltpu.SemaphoreType.DMA,
            pltpu.SemaphoreType.DMA,
        ],
    )
    def gather_kernel(v_hbm, idx_hbm, out_hbm, idx_v, out_v, sem_a, sem_b):
        wid = lax.axis_index("s") * _NC + lax.axis_index("c")
        base = wid * _B_PER_W
        idx_a = idx_v.at[pl.ds(0, _HALF)]
        idx_b = idx_v.at[pl.ds(_HALF, _HALF)]
        out_a = out_v.at[pl.ds(0, _HALF)]
        out_b = out_v.at[pl.ds(_HALF, _HALF)]

        pltpu.sync_copy(idx_hbm.at[pl.ds(base, _HALF)], idx_a)
        ga = pltpu.async_copy(v_hbm.at[idx_a], out_a, sem_a)
        pltpu.sync_copy(idx_hbm.at[pl.ds(base + _HALF, _HALF)], idx_b)
        gb = pltpu.async_copy(v_hbm.at[idx_b], out_b, sem_b)
        ga.wait()
        wa = pltpu.async_copy(out_a, out_hbm.at[pl.ds(base, _HALF)], sem_a)
        gb.wait()
        wb = pltpu.async_copy(out_b, out_hbm.at[pl.ds(base + _HALF, _HALF)],
                              sem_b)
        wa.wait()
        wb.wait()

    return gather_kernel


_GATHER = _make_gather()


def kernel(v, indices):
    return _GATHER(v, indices)


# speedup vs baseline: 1.0125x; 1.0125x over previous
import functools

import jax
import jax.numpy as jnp
from jax import lax
from jax.experimental import pallas as pl
from jax.experimental.pallas import tpu as pltpu
from jax.experimental.pallas import tpu_sc as plsc

_BATCH = 16384
_NC, _NS = 2, 16
_NW = _NC * _NS
_B_PER_W = _BATCH // _NW


def _make_gather():
    mesh = plsc.VectorSubcoreMesh(core_axis_name="c", subcore_axis_name="s")

    @functools.partial(
        pl.kernel,
        mesh=mesh,
        out_type=jax.ShapeDtypeStruct((_BATCH,), jnp.float32),
        scratch_types=[
            pltpu.VMEM((_B_PER_W,), jnp.int32),
            pltpu.VMEM((_B_PER_W,), jnp.float32),
            pltpu.SemaphoreType.DMA,
        ],
    )
    def gather_kernel(v_hbm, idx_hbm, out_hbm, idx_v, out_v, sem):
        wid = lax.axis_index("s") * _NC + lax.axis_index("c")
        base = wid * _B_PER_W
        pltpu.sync_copy(idx_hbm.at[pl.ds(base, _B_PER_W)], idx_v)
        pltpu.async_copy(v_hbm.at[idx_v], out_v, sem).wait()
        pltpu.sync_copy(out_v, out_hbm.at[pl.ds(base, _B_PER_W)])

    return gather_kernel


_GATHER = _make_gather()


def kernel(v, indices):
    return _GATHER(v, indices)


# speedup vs baseline: 1.0150x; 1.0025x over previous
import functools

import jax
import jax.numpy as jnp
from jax import lax
from jax.experimental import pallas as pl
from jax.experimental.pallas import tpu as pltpu
from jax.experimental.pallas import tpu_sc as plsc

_BATCH = 16384
_NC, _NS = 2, 16
_NW = _NC * _NS
_B_PER_W = _BATCH // _NW


def _make_gather():
    mesh = plsc.VectorSubcoreMesh(core_axis_name="c", subcore_axis_name="s")

    @functools.partial(
        pl.kernel,
        mesh=mesh,
        out_type=jax.ShapeDtypeStruct((_BATCH,), jnp.float32),
        scratch_types=[
            pltpu.VMEM((_B_PER_W,), jnp.int32),
            pltpu.VMEM((_B_PER_W,), jnp.float32),
            pltpu.SemaphoreType.DMA,
        ],
    )
    def gather_kernel(v_hbm, idx_hbm, out_hbm, idx_v, out_v, sem):
        wid = lax.axis_index("c") * _NS + lax.axis_index("s")
        base = wid * _B_PER_W
        pltpu.sync_copy(idx_hbm.at[pl.ds(base, _B_PER_W)], idx_v)
        pltpu.async_copy(v_hbm.at[idx_v], out_v, sem).wait()
        pltpu.sync_copy(out_v, out_hbm.at[pl.ds(base, _B_PER_W)])

    return gather_kernel


_GATHER = _make_gather()


def kernel(v, indices):
    return _GATHER(v, indices)


# speedup vs baseline: 1.0171x; 1.0021x over previous
import functools

import jax
import jax.numpy as jnp
from jax import lax
from jax.experimental import pallas as pl
from jax.experimental.pallas import tpu as pltpu
from jax.experimental.pallas import tpu_sc as plsc

_BATCH = 16384
_NC, _NS = 2, 16
_NW = _NC * _NS
_B_PER_W = _BATCH // _NW


def _make_gather():
    mesh = plsc.VectorSubcoreMesh(core_axis_name="c", subcore_axis_name="s")

    @functools.partial(
        pl.kernel,
        mesh=mesh,
        out_type=jax.ShapeDtypeStruct((_BATCH,), jnp.float32),
        scratch_types=[
            pltpu.VMEM((_B_PER_W,), jnp.int32),
            pltpu.VMEM((_B_PER_W,), jnp.float32),
            pltpu.SemaphoreType.DMA,
            pltpu.SemaphoreType.DMA,
            pltpu.SemaphoreType.DMA,
            pltpu.SemaphoreType.DMA,
        ],
    )
    def gather_kernel(
        v_hbm, idx_hbm, out_hbm, idx_v, out_v, g0, g1, s0, s1
    ):
        wid = lax.axis_index("c") * _NS + lax.axis_index("s")
        base = wid * _B_PER_W
        half = _B_PER_W // 2
        pltpu.sync_copy(idx_hbm.at[pl.ds(base, _B_PER_W)], idx_v)
        c0 = pltpu.async_copy(
            v_hbm.at[idx_v.at[pl.ds(0, half)]], out_v.at[pl.ds(0, half)], g0
        )
        c1 = pltpu.async_copy(
            v_hbm.at[idx_v.at[pl.ds(half, half)]],
            out_v.at[pl.ds(half, half)],
            g1,
        )
        c0.wait()
        w0 = pltpu.async_copy(
            out_v.at[pl.ds(0, half)], out_hbm.at[pl.ds(base, half)], s0
        )
        c1.wait()
        w1 = pltpu.async_copy(
            out_v.at[pl.ds(half, half)],
            out_hbm.at[pl.ds(base + half, half)],
            s1,
        )
        w0.wait()
        w1.wait()

    return gather_kernel


_GATHER = _make_gather()


def kernel(v, indices):
    return _GATHER(v, indices)
